# Initial kernel scaffold; baseline (speedup 1.0000x reference)
import jax
import jax.numpy as jnp
from jax.experimental import pallas as pl


def kernel(x, table):
    raise NotImplementedError("write your pallas kernel here")


# speedup vs baseline: 1.7452x; 1.7452x over previous
import functools
import math

import jax
import jax.numpy as jnp
from jax import lax
from jax.experimental import pallas as pl
from jax.experimental.pallas import tpu as pltpu
from jax.experimental.pallas import tpu_sc as plsc

D_MODEL = 128
SCALE = math.sqrt(float(D_MODEL))
NUM_CORES = 2
NUM_SUBCORES = 16
NUM_WORKERS = NUM_CORES * NUM_SUBCORES
LANES = 16
CHUNK = 128


def _scale_chunk(buf):
    def row_body(i, _):
        for j in range(D_MODEL // LANES):
            sl = pl.ds(j * LANES, LANES)
            buf[i, sl] = buf[i, sl] * SCALE
        return 0

    lax.fori_loop(0, CHUNK, row_body, 0)


@functools.partial(jax.jit, static_argnames=("n_chunks",))
def _embed_sc(x2d, table, n_chunks):
    rows_total = NUM_WORKERS * n_chunks * CHUNK
    mesh = plsc.VectorSubcoreMesh(core_axis_name="c", subcore_axis_name="s")

    @functools.partial(
        pl.kernel,
        mesh=mesh,
        out_type=jax.ShapeDtypeStruct((rows_total, D_MODEL), jnp.float32),
        scratch_types=[
            pltpu.VMEM((n_chunks, CHUNK), jnp.int32),
            pltpu.VMEM((CHUNK, D_MODEL), jnp.float32),
            pltpu.VMEM((CHUNK, D_MODEL), jnp.float32),
            pltpu.SemaphoreType.DMA,
            pltpu.SemaphoreType.DMA,
            pltpu.SemaphoreType.DMA,
            pltpu.SemaphoreType.DMA,
        ],
    )
    def k(x_hbm, table_hbm, out_hbm, idx_v, buf0, buf1, g0, g1, s0, s1):
        wid = lax.axis_index("s") * NUM_CORES + lax.axis_index("c")
        row_base = wid * (n_chunks * CHUNK)

        pltpu.sync_copy(x_hbm.at[pl.ds(wid * n_chunks, n_chunks)], idx_v)

        bufs = (buf0, buf1)
        gsems = (g0, g1)
        ssems = (s0, s1)

        def start_gather(g, p):
            pltpu.make_async_copy(
                table_hbm.at[idx_v.at[g]], bufs[p], gsems[p]
            ).start()

        def wait_gather(p):
            pltpu.make_async_copy(
                table_hbm.at[pl.ds(0, CHUNK)], bufs[p], gsems[p]
            ).wait()

        def start_scatter(g, p):
            pltpu.make_async_copy(
                bufs[p], out_hbm.at[pl.ds(row_base + g * CHUNK, CHUNK)], ssems[p]
            ).start()

        def wait_scatter(p):
            pltpu.make_async_copy(
                bufs[p], out_hbm.at[pl.ds(row_base, CHUNK)], ssems[p]
            ).wait()

        start_gather(0, 0)

        def loop_body(gg, _):
            for b in range(2):
                g = gg + b
                p = b
                q = 1 - b
                if b == 0:
                    @pl.when(gg > 0)
                    def _():
                        wait_scatter(q)
                        start_gather(g + 1, q)

                    @pl.when(gg == 0)
                    def _():
                        start_gather(g + 1, q)
                else:
                    @pl.when(gg < n_chunks - 2)
                    def _():
                        wait_scatter(q)
                        start_gather(g + 1, q)
                wait_gather(p)
                _scale_chunk(bufs[p])
                start_scatter(g, p)
            return 0

        lax.fori_loop(0, n_chunks // 2, lambda t, c: loop_body(t * 2, c), 0)

        wait_scatter(0)
        wait_scatter(1)

    return k(x2d, table)


def kernel(x, table):
    seq_shape = x.shape
    n_idx = x.size
    assert n_idx % (NUM_WORKERS * CHUNK) == 0
    n_chunks = n_idx // (NUM_WORKERS * CHUNK)
    x2d = jnp.reshape(x.astype(jnp.int32), (NUM_WORKERS * n_chunks, CHUNK))
    out = _embed_sc(x2d, table, n_chunks)
    return jnp.reshape(out, seq_shape + (D_MODEL,))


# speedup vs baseline: 1.8611x; 1.0664x over previous
import functools
import math

import jax
import jax.numpy as jnp
from jax import lax
from jax.experimental import pallas as pl
from jax.experimental.pallas import tpu as pltpu
from jax.experimental.pallas import tpu_sc as plsc

D_MODEL = 128
SCALE = math.sqrt(float(D_MODEL))
NUM_CORES = 2
NUM_SUBCORES = 16
NUM_WORKERS = NUM_CORES * NUM_SUBCORES
LANES = 16
CHUNK = 128


def _scale_chunk(buf):
    def row_body(i, _):
        for j in range(D_MODEL // LANES):
            sl = pl.ds(j * LANES, LANES)
            buf[i, sl] = buf[i, sl] * SCALE
        return 0

    lax.fori_loop(0, CHUNK, row_body, 0)


@functools.partial(jax.jit, static_argnames=("n_chunks",))
def _embed_sc(x2d, table, n_chunks):
    rows_total = NUM_WORKERS * n_chunks * CHUNK
    mesh = plsc.VectorSubcoreMesh(core_axis_name="c", subcore_axis_name="s")
    NBUF = 4

    @functools.partial(
        pl.kernel,
        mesh=mesh,
        out_type=jax.ShapeDtypeStruct((rows_total, D_MODEL), jnp.float32),
        scratch_types=[
            pltpu.VMEM((n_chunks, CHUNK), jnp.int32),
        ]
        + [pltpu.VMEM((CHUNK, D_MODEL), jnp.float32)] * NBUF
        + [pltpu.SemaphoreType.DMA] * (2 * NBUF),
    )
    def k(x_hbm, table_hbm, out_hbm, idx_v, *bufs_sems):
        bufs = bufs_sems[:NBUF]
        gsems = bufs_sems[NBUF : 2 * NBUF]
        ssems = bufs_sems[2 * NBUF :]
        wid = lax.axis_index("s") * NUM_CORES + lax.axis_index("c")
        row_base = wid * (n_chunks * CHUNK)

        pltpu.sync_copy(x_hbm.at[pl.ds(wid * n_chunks, n_chunks)], idx_v)

        def start_gather(g, p):
            pltpu.make_async_copy(
                table_hbm.at[idx_v.at[g]], bufs[p], gsems[p]
            ).start()

        def wait_gather(p):
            pltpu.make_async_copy(
                table_hbm.at[pl.ds(0, CHUNK)], bufs[p], gsems[p]
            ).wait()

        def start_scatter(g, p):
            pltpu.make_async_copy(
                bufs[p], out_hbm.at[pl.ds(row_base + g * CHUNK, CHUNK)], ssems[p]
            ).start()

        def wait_scatter(p):
            pltpu.make_async_copy(
                bufs[p], out_hbm.at[pl.ds(row_base, CHUNK)], ssems[p]
            ).wait()

        start_gather(0, 0)
        start_gather(1, 1)

        def loop_body(gg, _):
            for b in range(NBUF):
                g = gg + b
                p = b
                pf = (b + 2) % NBUF
                if b < 2:
                    @pl.when(gg > 0)
                    def _():
                        wait_scatter(pf)

                    start_gather(g + 2, pf)
                else:
                    @pl.when(gg < n_chunks - NBUF)
                    def _():
                        wait_scatter(pf)
                        start_gather(g + 2, pf)

                wait_gather(p)
                _scale_chunk(bufs[p])
                start_scatter(g, p)
            return 0

        lax.fori_loop(0, n_chunks // NBUF, lambda t, c: loop_body(t * NBUF, c), 0)

        for p in range(NBUF):
            wait_scatter(p)

    return k(x2d, table)


def kernel(x, table):
    seq_shape = x.shape
    n_idx = x.size
    assert n_idx % (NUM_WORKERS * CHUNK) == 0
    n_chunks = n_idx // (NUM_WORKERS * CHUNK)
    x2d = jnp.reshape(x.astype(jnp.int32), (NUM_WORKERS * n_chunks, CHUNK))
    out = _embed_sc(x2d, table, n_chunks)
    return jnp.reshape(out, seq_shape + (D_MODEL,))
